# initial kernel scaffold (unmeasured)
import jax
import jax.numpy as jnp
from jax import lax
from jax.experimental import pallas as pl
from jax.experimental.pallas import tpu as pltpu

N_DEV = 8
B, H, D, BS = 16, 16, 64, 16
NPAGES = 128
NKEYS = NPAGES * BS
NEG = -1e30


def kernel(Q, K, V, bt, lens):
    def body(q_ref, k_ref, v_ref, bt_ref, lens_ref, out_ref,
             gather_ref, send_sems, recv_sems):
        my = lax.axis_index("i")
        left = lax.rem(my + N_DEV - 1, N_DEV)
        right = lax.rem(my + 1, N_DEV)

        barrier = pltpu.get_barrier_semaphore()
        for nbr in (left, right):
            pl.semaphore_signal(barrier, inc=1, device_id=(nbr,),
                                device_id_type=pl.DeviceIdType.MESH)
        pl.semaphore_wait(barrier, 2)

        page_base = my * NPAGES
        li = bt_ref[...] - page_base
        lens_v = lens_ref[...].reshape(B, 1)
        p_iota = lax.broadcasted_iota(jnp.int32, (B, NPAGES), 1)
        in_range = (li >= 0) & (li < NPAGES) & (p_iota < lens_v)
        kp = lax.broadcasted_iota(jnp.int32, (B, NPAGES, NKEYS), 2) // BS
        match = (li[:, :, None] == kp) & in_range[:, :, None]
        w = jnp.sum(jnp.where(match, 1.0, 0.0), axis=1)
        wpos = w > 0.0

        scale = D ** -0.5
        q_all = q_ref[:, 0, :, :].astype(jnp.bfloat16)
        for h in range(H):
            kh = k_ref[:, :, h, :].reshape(NKEYS, D).astype(jnp.bfloat16)
            vh = v_ref[:, :, h, :].reshape(NKEYS, D).astype(jnp.bfloat16)
            qh = q_all[:, h, :]
            s = lax.dot_general(qh, kh, (((1,), (1,)), ((), ())),
                                preferred_element_type=jnp.float32) * scale
            s = jnp.where(wpos, s, NEG)
            m = jnp.max(s, axis=1, keepdims=True)
            p = w * jnp.exp(s - m)
            l = jnp.sum(p, axis=1, keepdims=True)
            acc = lax.dot_general(p.astype(jnp.bfloat16), vh,
                                  (((1,), (0,)), ((), ())),
                                  preferred_element_type=jnp.float32)
            gather_ref[0, :, h, 0:D] = acc
            gather_ref[0, :, h, D:D + 1] = m
            gather_ref[0, :, h, D + 1:D + 2] = l

        for hop in range(N_DEV - 1):
            rdma = pltpu.make_async_remote_copy(
                src_ref=gather_ref.at[hop],
                dst_ref=gather_ref.at[hop + 1],
                send_sem=send_sems.at[hop],
                recv_sem=recv_sems.at[hop],
                device_id=(right,),
                device_id_type=pl.DeviceIdType.MESH,
            )
            rdma.start()
            rdma.wait()

        chunks = gather_ref[...]
        acc = chunks[:, :, :, 0:D]
        m = chunks[:, :, :, D:D + 1]
        l = chunks[:, :, :, D + 1:D + 2]
        m_tot = jnp.max(m, axis=0)
        sc = jnp.exp(m - m_tot)
        l_tot = jnp.sum(l * sc, axis=0)
        out = jnp.sum(acc * sc, axis=0) / l_tot
        out_ref[:, 0, :, :] = out

    return pl.pallas_call(
        body,
        out_shape=jax.ShapeDtypeStruct((B, 1, H, D), jnp.float32),
        in_specs=[pl.BlockSpec(memory_space=pltpu.VMEM)] * 5,
        out_specs=pl.BlockSpec(memory_space=pltpu.VMEM),
        scratch_shapes=[
            pltpu.VMEM((N_DEV, B, H, 128), jnp.float32),
            pltpu.SemaphoreType.DMA((N_DEV - 1,)),
            pltpu.SemaphoreType.DMA((N_DEV - 1,)),
        ],
        compiler_params=pltpu.CompilerParams(collective_id=0),
    )(Q, K, V, bt, lens)


# baseline (device time: 111736 ns/iter reference)
import jax
import jax.numpy as jnp
from jax import lax
from jax.experimental import pallas as pl
from jax.experimental.pallas import tpu as pltpu

N_DEV = 8
B, H, D, BS = 16, 16, 64, 16
NPAGES = 128
NKEYS = NPAGES * BS
NEG = -1e30


def kernel(Q, K, V, bt, lens):
    def body(q_ref, k_ref, v_ref, bt_ref, lens_ref, out_ref,
             gather_ref, send_sems, recv_sems):
        my = lax.axis_index("i")
        left = lax.rem(my + N_DEV - 1, N_DEV)
        right = lax.rem(my + 1, N_DEV)

        barrier = pltpu.get_barrier_semaphore()
        for nbr in (left, right):
            pl.semaphore_signal(barrier, inc=1, device_id=(nbr,),
                                device_id_type=pl.DeviceIdType.MESH)
        pl.semaphore_wait(barrier, 2)

        page_base = my * NPAGES
        li = bt_ref[...] - page_base
        lens_v = lens_ref[...].reshape(B, 1)
        p_iota = lax.broadcasted_iota(jnp.int32, (B, NPAGES), 1)
        in_range = (li >= 0) & (li < NPAGES) & (p_iota < lens_v)
        li1 = jnp.where(in_range, li, -1)
        kp = lax.broadcasted_iota(jnp.int32, (B, NPAGES, NKEYS), 2) // BS
        match = li1[:, :, None] == kp
        w = jnp.sum(jnp.where(match, 1.0, 0.0), axis=1)
        wpos = w > 0.0

        scale = D ** -0.5
        q_all = q_ref[:, 0, :, :].astype(jnp.bfloat16)
        for h in range(H):
            kh = k_ref[:, :, h, :].reshape(NKEYS, D).astype(jnp.bfloat16)
            vh = v_ref[:, :, h, :].reshape(NKEYS, D).astype(jnp.bfloat16)
            qh = q_all[:, h, :]
            s = lax.dot_general(qh, kh, (((1,), (1,)), ((), ())),
                                preferred_element_type=jnp.float32) * scale
            s = jnp.where(wpos, s, NEG)
            m = jnp.max(s, axis=1, keepdims=True)
            p = w * jnp.exp(s - m)
            l = jnp.sum(p, axis=1, keepdims=True)
            acc = lax.dot_general(p.astype(jnp.bfloat16), vh,
                                  (((1,), (0,)), ((), ())),
                                  preferred_element_type=jnp.float32)
            gather_ref[0, :, h, 0:D] = acc
            gather_ref[0, :, h, D:D + 1] = m
            gather_ref[0, :, h, D + 1:D + 2] = l

        for hop in range(N_DEV - 1):
            rdma = pltpu.make_async_remote_copy(
                src_ref=gather_ref.at[hop],
                dst_ref=gather_ref.at[hop + 1],
                send_sem=send_sems.at[hop],
                recv_sem=recv_sems.at[hop],
                device_id=(right,),
                device_id_type=pl.DeviceIdType.MESH,
            )
            rdma.start()
            rdma.wait()

        chunks = gather_ref[...]
        acc = chunks[:, :, :, 0:D]
        m = chunks[:, :, :, D:D + 1]
        l = chunks[:, :, :, D + 1:D + 2]
        m_tot = jnp.max(m, axis=0)
        sc = jnp.exp(m - m_tot)
        l_tot = jnp.sum(l * sc, axis=0)
        out = jnp.sum(acc * sc, axis=0) / l_tot
        out_ref[:, 0, :, :] = out

    return pl.pallas_call(
        body,
        out_shape=jax.ShapeDtypeStruct((B, 1, H, D), jnp.float32),
        in_specs=[pl.BlockSpec(memory_space=pltpu.VMEM)] * 5,
        out_specs=pl.BlockSpec(memory_space=pltpu.VMEM),
        scratch_shapes=[
            pltpu.VMEM((N_DEV, B, H, 128), jnp.float32),
            pltpu.SemaphoreType.DMA((N_DEV - 1,)),
            pltpu.SemaphoreType.DMA((N_DEV - 1,)),
        ],
        compiler_params=pltpu.CompilerParams(collective_id=0),
    )(Q, K, V, bt, lens)


# device time: 43381 ns/iter; 2.5757x vs baseline; 2.5757x over previous
import jax
import jax.numpy as jnp
from jax import lax
from jax.experimental import pallas as pl
from jax.experimental.pallas import tpu as pltpu

N_DEV = 8
B, H, D, BS = 16, 16, 64, 16
NPAGES = 128
NEG = -1e30


def kernel(Q, K, V, bt, lens):
    Kt = jnp.transpose(K, (1, 2, 3, 0))
    Vt = jnp.transpose(V, (1, 2, 3, 0))

    def body(q_ref, k_ref, v_ref, bt_ref, lens_ref, out_ref,
             gather_ref, send_sems, recv_sems):
        my = lax.axis_index("i")
        left = lax.rem(my + N_DEV - 1, N_DEV)
        right = lax.rem(my + 1, N_DEV)

        barrier = pltpu.get_barrier_semaphore()
        for nbr in (left, right):
            pl.semaphore_signal(barrier, inc=1, device_id=(nbr,),
                                device_id_type=pl.DeviceIdType.MESH)
        pl.semaphore_wait(barrier, 2)

        page_base = my * NPAGES
        li = bt_ref[...] - page_base
        lens_v = lens_ref[...].reshape(B, 1)
        p_iota = lax.broadcasted_iota(jnp.int32, (B, NPAGES), 1)
        in_range = (li >= 0) & (li < NPAGES) & (p_iota < lens_v)
        li1 = jnp.where(in_range, li, -1)
        g_iota = lax.broadcasted_iota(jnp.int32, (B, NPAGES, NPAGES), 1)
        match = li1.reshape(B, 1, NPAGES) == g_iota
        w = jnp.sum(jnp.where(match, 1.0, 0.0), axis=2)
        bias = jnp.where(w > 0.0, jnp.log(jnp.maximum(w, 1e-30)), NEG)
        bias3 = jnp.broadcast_to(bias.reshape(1, B, NPAGES), (BS, B, NPAGES))

        scale = D ** -0.5
        kb = k_ref[...].astype(jnp.bfloat16)
        vb = v_ref[...].astype(jnp.bfloat16)
        q_all = q_ref[:, 0, :, :].astype(jnp.bfloat16)
        for h in range(H):
            kh = kb[:, h]
            vh = vb[:, h]
            qh = jnp.broadcast_to(q_all[:, h, :].reshape(1, B, D), (BS, B, D))
            s = lax.dot_general(qh, kh, (((2,), (1,)), ((0,), (0,))),
                                preferred_element_type=jnp.float32) * scale
            s = s + bias3
            m = jnp.max(jnp.max(s, axis=0), axis=1, keepdims=True)
            p = jnp.exp(s - m)
            l = jnp.sum(jnp.sum(p, axis=0), axis=1, keepdims=True)
            acc = jnp.sum(
                lax.dot_general(p.astype(jnp.bfloat16), vh,
                                (((2,), (2,)), ((0,), (0,))),
                                preferred_element_type=jnp.float32),
                axis=0)
            gather_ref[0, :, h, 0:D] = acc
            gather_ref[0, :, h, D:D + 1] = m
            gather_ref[0, :, h, D + 1:D + 2] = l

        for hop in range(N_DEV - 1):
            rdma = pltpu.make_async_remote_copy(
                src_ref=gather_ref.at[hop],
                dst_ref=gather_ref.at[hop + 1],
                send_sem=send_sems.at[hop],
                recv_sem=recv_sems.at[hop],
                device_id=(right,),
                device_id_type=pl.DeviceIdType.MESH,
            )
            rdma.start()
            rdma.wait()

        chunks = gather_ref[...]
        acc = chunks[:, :, :, 0:D]
        m = chunks[:, :, :, D:D + 1]
        l = chunks[:, :, :, D + 1:D + 2]
        m_tot = jnp.max(m, axis=0)
        sc = jnp.exp(m - m_tot)
        l_tot = jnp.sum(l * sc, axis=0)
        out = jnp.sum(acc * sc, axis=0) / l_tot
        out_ref[:, 0, :, :] = out

    return pl.pallas_call(
        body,
        out_shape=jax.ShapeDtypeStruct((B, 1, H, D), jnp.float32),
        in_specs=[pl.BlockSpec(memory_space=pltpu.VMEM)] * 5,
        out_specs=pl.BlockSpec(memory_space=pltpu.VMEM),
        scratch_shapes=[
            pltpu.VMEM((N_DEV, B, H, 128), jnp.float32),
            pltpu.SemaphoreType.DMA((N_DEV - 1,)),
            pltpu.SemaphoreType.DMA((N_DEV - 1,)),
        ],
        compiler_params=pltpu.CompilerParams(collective_id=0),
    )(Q, Kt, Vt, bt, lens)


# device time: 29594 ns/iter; 3.7756x vs baseline; 1.4659x over previous
import jax
import jax.numpy as jnp
from jax import lax
from jax.experimental import pallas as pl
from jax.experimental.pallas import tpu as pltpu

N_DEV = 8
B, H, D, BS = 16, 16, 64, 16
NPAGES = 128
NEG = -1e30


def kernel(Q, K, V, bt, lens):
    Kt = jnp.transpose(K, (1, 2, 3, 0))
    Vt = jnp.transpose(V, (1, 2, 3, 0))

    def body(q_ref, k_ref, v_ref, bt_ref, lens_ref, out_ref,
             gather_ref, send_sems, recv_sems):
        my = lax.axis_index("i")

        barrier = pltpu.get_barrier_semaphore()
        for k in range(1, N_DEV):
            pl.semaphore_signal(barrier, inc=1,
                                device_id=(lax.rem(my + k, N_DEV),),
                                device_id_type=pl.DeviceIdType.MESH)
        pl.semaphore_wait(barrier, N_DEV - 1)

        page_base = my * NPAGES
        li = bt_ref[...] - page_base
        lens_v = lens_ref[...].reshape(B, 1)
        p_iota = lax.broadcasted_iota(jnp.int32, (B, NPAGES), 1)
        in_range = (li >= 0) & (li < NPAGES) & (p_iota < lens_v)
        li1 = jnp.where(in_range, li, -1)
        g_iota = lax.broadcasted_iota(jnp.int32, (B, NPAGES, NPAGES), 1)
        match = li1.reshape(B, 1, NPAGES) == g_iota
        w = jnp.sum(jnp.where(match, 1.0, 0.0), axis=2)
        bias = jnp.where(w > 0.0, jnp.log(jnp.maximum(w, 1e-30)), NEG)
        bias3 = jnp.broadcast_to(bias.reshape(1, B, NPAGES), (BS, B, NPAGES))

        scale = D ** -0.5
        kb = k_ref[...].astype(jnp.bfloat16)
        vb = v_ref[...].astype(jnp.bfloat16)
        q_all = q_ref[:, 0, :, :].astype(jnp.bfloat16)
        for h in range(H):
            kh = kb[:, h]
            vh = vb[:, h]
            qh = jnp.broadcast_to(q_all[:, h, :].reshape(1, B, D), (BS, B, D))
            s = lax.dot_general(qh, kh, (((2,), (1,)), ((0,), (0,))),
                                preferred_element_type=jnp.float32) * scale
            s = s + bias3
            m = jnp.max(jnp.max(s, axis=0), axis=1, keepdims=True)
            p = jnp.exp(s - m)
            l = jnp.sum(jnp.sum(p, axis=0), axis=1, keepdims=True)
            acc = jnp.sum(
                lax.dot_general(p.astype(jnp.bfloat16), vh,
                                (((2,), (2,)), ((0,), (0,))),
                                preferred_element_type=jnp.float32),
                axis=0)
            gather_ref[0, :, h, 0:D] = acc
            gather_ref[0, :, h, D:D + 1] = m
            gather_ref[0, :, h, D + 1:D + 2] = l

        rdmas = []
        for k in range(1, N_DEV):
            rdma = pltpu.make_async_remote_copy(
                src_ref=gather_ref.at[0],
                dst_ref=gather_ref.at[k],
                send_sem=send_sems.at[k - 1],
                recv_sem=recv_sems.at[k - 1],
                device_id=(lax.rem(my + k, N_DEV),),
                device_id_type=pl.DeviceIdType.MESH,
            )
            rdma.start()
            rdmas.append(rdma)
        for rdma in rdmas:
            rdma.wait()

        chunks = gather_ref[...]
        acc = chunks[:, :, :, 0:D]
        m = chunks[:, :, :, D:D + 1]
        l = chunks[:, :, :, D + 1:D + 2]
        m_tot = jnp.max(m, axis=0)
        sc = jnp.exp(m - m_tot)
        l_tot = jnp.sum(l * sc, axis=0)
        out = jnp.sum(acc * sc, axis=0) / l_tot
        out_ref[:, 0, :, :] = out

    return pl.pallas_call(
        body,
        out_shape=jax.ShapeDtypeStruct((B, 1, H, D), jnp.float32),
        in_specs=[pl.BlockSpec(memory_space=pltpu.VMEM)] * 5,
        out_specs=pl.BlockSpec(memory_space=pltpu.VMEM),
        scratch_shapes=[
            pltpu.VMEM((N_DEV, B, H, 128), jnp.float32),
            pltpu.SemaphoreType.DMA((N_DEV - 1,)),
            pltpu.SemaphoreType.DMA((N_DEV - 1,)),
        ],
        compiler_params=pltpu.CompilerParams(collective_id=0),
    )(Q, Kt, Vt, bt, lens)


# device time: 27012 ns/iter; 4.1365x vs baseline; 1.0956x over previous
import jax
import jax.numpy as jnp
from jax import lax
from jax.experimental import pallas as pl
from jax.experimental.pallas import tpu as pltpu

N_DEV = 8
B, H, D, BS = 16, 16, 64, 16
NPAGES = 128
NEG = -1e30


def kernel(Q, K, V, bt, lens):
    Kt = jnp.transpose(K, (1, 2, 3, 0))
    Vt = jnp.transpose(V, (1, 2, 3, 0))

    def body(q_ref, k_ref, v_ref, bt_ref, lens_ref, out_ref,
             gather_ref, send_sems, recv_sems):
        my = lax.axis_index("i")

        barrier = pltpu.get_barrier_semaphore()
        for k in range(1, N_DEV):
            pl.semaphore_signal(barrier, inc=1,
                                device_id=(lax.rem(my + k, N_DEV),),
                                device_id_type=pl.DeviceIdType.MESH)
        pl.semaphore_wait(barrier, N_DEV - 1)

        page_base = my * NPAGES
        li = bt_ref[...] - page_base
        lens_v = lens_ref[...].reshape(B, 1)
        p_iota = lax.broadcasted_iota(jnp.int32, (B, NPAGES), 1)
        in_range = (li >= 0) & (li < NPAGES) & (p_iota < lens_v)
        li1 = jnp.where(in_range, li, -1)
        g_iota = lax.broadcasted_iota(jnp.int32, (B, NPAGES, NPAGES), 1)
        match = li1.reshape(B, 1, NPAGES) == g_iota
        w = jnp.sum(jnp.where(match, 1.0, 0.0), axis=2)
        bias = jnp.where(w > 0.0, jnp.log(jnp.maximum(w, 1e-30)), NEG)
        bias3 = jnp.broadcast_to(bias.reshape(1, B, NPAGES), (BS, B, NPAGES))

        scale = D ** -0.5
        kb = k_ref[...].astype(jnp.bfloat16)
        vb = v_ref[...].astype(jnp.bfloat16)
        q_all = q_ref[:, 0, :, :].astype(jnp.bfloat16)
        for h in range(H):
            kh = kb[:, h]
            vh = vb[:, h]
            qh = jnp.broadcast_to(q_all[:, h, :].reshape(1, B, D), (BS, B, D))
            s = lax.dot_general(qh, kh, (((2,), (1,)), ((0,), (0,))),
                                preferred_element_type=jnp.float32) * scale
            s = s + bias3
            m = jnp.max(jnp.max(s, axis=0), axis=1, keepdims=True)
            m = m.astype(jnp.bfloat16).astype(jnp.float32)
            p = jnp.exp(s - m)
            l = jnp.sum(jnp.sum(p, axis=0), axis=1, keepdims=True)
            acc = jnp.sum(
                lax.dot_general(p.astype(jnp.bfloat16), vh,
                                (((2,), (2,)), ((0,), (0,))),
                                preferred_element_type=jnp.float32),
                axis=0)
            gather_ref[0, :, h, 0:D] = acc.astype(jnp.bfloat16)
            gather_ref[0, :, h, D:D + 1] = m.astype(jnp.bfloat16)
            gather_ref[0, :, h, D + 1:D + 2] = l.astype(jnp.bfloat16)

        rdmas = []
        for k in range(1, N_DEV):
            rdma = pltpu.make_async_remote_copy(
                src_ref=gather_ref.at[0],
                dst_ref=gather_ref.at[k],
                send_sem=send_sems.at[k - 1],
                recv_sem=recv_sems.at[k - 1],
                device_id=(lax.rem(my + k, N_DEV),),
                device_id_type=pl.DeviceIdType.MESH,
            )
            rdma.start()
            rdmas.append(rdma)
        for rdma in rdmas:
            rdma.wait()

        chunks = gather_ref[...].astype(jnp.float32)
        acc = chunks[:, :, :, 0:D]
        m = chunks[:, :, :, D:D + 1]
        l = chunks[:, :, :, D + 1:D + 2]
        m_tot = jnp.max(m, axis=0)
        sc = jnp.exp(m - m_tot)
        l_tot = jnp.sum(l * sc, axis=0)
        out = jnp.sum(acc * sc, axis=0) / l_tot
        out_ref[:, 0, :, :] = out

    return pl.pallas_call(
        body,
        out_shape=jax.ShapeDtypeStruct((B, 1, H, D), jnp.float32),
        in_specs=[pl.BlockSpec(memory_space=pltpu.VMEM)] * 5,
        out_specs=pl.BlockSpec(memory_space=pltpu.VMEM),
        scratch_shapes=[
            pltpu.VMEM((N_DEV, B, H, 128), jnp.bfloat16),
            pltpu.SemaphoreType.DMA((N_DEV - 1,)),
            pltpu.SemaphoreType.DMA((N_DEV - 1,)),
        ],
        compiler_params=pltpu.CompilerParams(collective_id=0),
    )(Q, Kt, Vt, bt, lens)
